# baseline (speedup 1.0000x reference)
import jax
import jax.numpy as jnp
from jax import lax
from jax.experimental import pallas as pl
from jax.experimental.pallas import tpu as pltpu
from jax.experimental.pallas import tpu_sc as plsc

BATCH = 4096
CTX = 200
VOCAB = 100000
DIM = 64
N = BATCH * CTX
NC = 2
NS = 16
NW = NC * NS
PER_W = N // NW
CHUNK = 128
N_CHUNKS = PER_W // CHUNK
NBUF = 4
G = N_CHUNKS // NBUF


def _emb_body(xp_hbm, emb_hbm, pos_hbm, out_hbm, xpib, ebuf, pbuf, obuf,
              isem, gsem, wsem):
    wid = lax.axis_index("s") * NC + lax.axis_index("c")

    def issue_gathers(b, q):
        pltpu.async_copy(emb_hbm.at[xpib.at[b, q, 0]], ebuf.at[b], gsem.at[b])
        pltpu.async_copy(pos_hbm.at[xpib.at[b, q, 1]], pbuf.at[b], gsem.at[b])

    def wait_gathers(b):
        pltpu.make_async_copy(emb_hbm.at[pl.ds(0, CHUNK)], ebuf.at[b],
                              gsem.at[b]).wait()
        pltpu.make_async_copy(pos_hbm.at[pl.ds(0, CHUNK)], pbuf.at[b],
                              gsem.at[b]).wait()

    for b in range(NBUF):
        pltpu.sync_copy(xp_hbm.at[wid, b], xpib.at[b, 0])
    for b in range(NBUF):
        issue_gathers(b, 0)
    for b in range(NBUF):
        pltpu.async_copy(xp_hbm.at[wid, NBUF + b], xpib.at[b, 1], isem.at[b])

    def outer(g, _):
        for b in range(NBUF):
            c = g * NBUF + b

            @pl.when(g > 0)
            def _():
                pltpu.make_async_copy(obuf.at[b], out_hbm.at[wid, 0],
                                      wsem.at[b]).wait()

            wait_gathers(b)

            def row_body(i, _):
                for j in range(DIM // 16):
                    s = pl.ds(j * 16, 16)
                    obuf[b, i, s] = ebuf[b, i, s] + pbuf[b, i, s]
                return 0

            lax.fori_loop(0, CHUNK, row_body, 0)

            pltpu.async_copy(obuf.at[b], out_hbm.at[wid, c], wsem.at[b])

            @pl.when(g < G - 1)
            def _():
                pltpu.make_async_copy(xp_hbm.at[wid, 0], xpib.at[b, 0],
                                      isem.at[b]).wait()
                issue_gathers(b, (g + 1) % 2)

            @pl.when(g < G - 2)
            def _():
                pltpu.async_copy(xp_hbm.at[wid, c + 2 * NBUF],
                                 xpib.at[b, g % 2], isem.at[b])

        return 0

    lax.fori_loop(0, G, outer, 0)

    for b in range(NBUF):
        pltpu.make_async_copy(obuf.at[b], out_hbm.at[wid, 0], wsem.at[b]).wait()


@jax.jit
def kernel(x, _pos, emb_table, pos_table):
    xf = x.reshape(NW, N_CHUNKS, 1, CHUNK).astype(jnp.int32)
    pf = _pos.reshape(NW, N_CHUNKS, 1, CHUNK).astype(jnp.int32)
    xp = jnp.concatenate([xf, pf], axis=2)
    k = pl.kernel(
        _emb_body,
        out_type=jax.ShapeDtypeStruct((NW, N_CHUNKS, CHUNK, DIM), jnp.float32),
        mesh=plsc.VectorSubcoreMesh(core_axis_name="c", subcore_axis_name="s"),
        compiler_params=pltpu.CompilerParams(use_tc_tiling_on_sc=False),
        scratch_types=[
            pltpu.VMEM((NBUF, 2, 2, CHUNK), jnp.int32),
            pltpu.VMEM((NBUF, CHUNK, DIM), jnp.float32),
            pltpu.VMEM((NBUF, CHUNK, DIM), jnp.float32),
            pltpu.VMEM((NBUF, CHUNK, DIM), jnp.float32),
            pltpu.SemaphoreType.DMA((NBUF,)),
            pltpu.SemaphoreType.DMA((NBUF,)),
            pltpu.SemaphoreType.DMA((NBUF,)),
        ],
    )
    out = k(xp, emb_table, pos_table)
    return out.reshape(BATCH, CTX, DIM)


# speedup vs baseline: 1.0136x; 1.0136x over previous
import jax
import jax.numpy as jnp
from jax import lax
from jax.experimental import pallas as pl
from jax.experimental.pallas import tpu as pltpu
from jax.experimental.pallas import tpu_sc as plsc

BATCH = 4096
CTX = 200
VOCAB = 100000
DIM = 64
N = BATCH * CTX
NC = 2
NS = 16
NW = NC * NS
PER_W = N // NW
CHUNK = 128
N_CHUNKS = PER_W // CHUNK
NBUF = 4
G = N_CHUNKS // NBUF


def _emb_body(x_hbm, p_hbm, emb_hbm, pos_hbm, out_hbm, xib, pib, ebuf, pbuf,
              obuf, isem, gsem, wsem):
    wid = lax.axis_index("s") * NC + lax.axis_index("c")

    def issue_gathers(b, q):
        pltpu.async_copy(emb_hbm.at[xib.at[b, q]], ebuf.at[b], gsem.at[b])
        pltpu.async_copy(pos_hbm.at[pib.at[b, q]], pbuf.at[b], gsem.at[b])

    def wait_gathers(b):
        pltpu.make_async_copy(emb_hbm.at[pl.ds(0, CHUNK)], ebuf.at[b],
                              gsem.at[b]).wait()
        pltpu.make_async_copy(pos_hbm.at[pl.ds(0, CHUNK)], pbuf.at[b],
                              gsem.at[b]).wait()

    def issue_idx(b, c, q):
        pltpu.async_copy(x_hbm.at[wid, c], xib.at[b, q], isem.at[b])
        pltpu.async_copy(p_hbm.at[wid, c], pib.at[b, q], isem.at[b])

    def wait_idx(b):
        pltpu.make_async_copy(x_hbm.at[wid, 0], xib.at[b, 0], isem.at[b]).wait()
        pltpu.make_async_copy(p_hbm.at[wid, 0], pib.at[b, 0], isem.at[b]).wait()

    for b in range(NBUF):
        pltpu.sync_copy(x_hbm.at[wid, b], xib.at[b, 0])
        pltpu.sync_copy(p_hbm.at[wid, b], pib.at[b, 0])
    for b in range(NBUF):
        issue_gathers(b, 0)
    for b in range(NBUF):
        issue_idx(b, NBUF + b, 1)

    def outer(g, _):
        for b in range(NBUF):
            c = g * NBUF + b

            @pl.when(g > 0)
            def _():
                pltpu.make_async_copy(obuf.at[b], out_hbm.at[wid, 0],
                                      wsem.at[b]).wait()

            wait_gathers(b)

            def row_body(i, _):
                for j in range(DIM // 16):
                    s = pl.ds(j * 16, 16)
                    obuf[b, i, s] = ebuf[b, i, s] + pbuf[b, i, s]
                return 0

            lax.fori_loop(0, CHUNK, row_body, 0)

            pltpu.async_copy(obuf.at[b], out_hbm.at[wid, c], wsem.at[b])

            @pl.when(g < G - 1)
            def _():
                wait_idx(b)
                issue_gathers(b, (g + 1) % 2)

            @pl.when(g < G - 2)
            def _():
                issue_idx(b, c + 2 * NBUF, g % 2)

        return 0

    lax.fori_loop(0, G, outer, 0)

    for b in range(NBUF):
        pltpu.make_async_copy(obuf.at[b], out_hbm.at[wid, 0], wsem.at[b]).wait()


@jax.jit
def kernel(x, _pos, emb_table, pos_table):
    xf = x.reshape(NW, N_CHUNKS, CHUNK)
    pf = _pos.reshape(NW, N_CHUNKS, CHUNK)
    k = pl.kernel(
        _emb_body,
        out_type=jax.ShapeDtypeStruct((NW, N_CHUNKS, CHUNK, DIM), jnp.float32),
        mesh=plsc.VectorSubcoreMesh(core_axis_name="c", subcore_axis_name="s"),
        compiler_params=pltpu.CompilerParams(use_tc_tiling_on_sc=False),
        scratch_types=[
            pltpu.VMEM((NBUF, 2, CHUNK), jnp.int32),
            pltpu.VMEM((NBUF, 2, CHUNK), jnp.int32),
            pltpu.VMEM((NBUF, CHUNK, DIM), jnp.float32),
            pltpu.VMEM((NBUF, CHUNK, DIM), jnp.float32),
            pltpu.VMEM((NBUF, CHUNK, DIM), jnp.float32),
            pltpu.SemaphoreType.DMA((NBUF,)),
            pltpu.SemaphoreType.DMA((NBUF,)),
            pltpu.SemaphoreType.DMA((NBUF,)),
        ],
    )
    out = k(xf, pf, emb_table, pos_table)
    return out.reshape(BATCH, CTX, DIM)
